# baseline (speedup 1.0000x reference)
import functools

import jax
import jax.numpy as jnp
from jax import lax
from jax.experimental import pallas as pl
from jax.experimental.pallas import tpu as pltpu, tpu_sc as plsc

BATCH = 16384
FIELDS = 100
N_IN = BATCH * FIELDS
N_WORKERS = 32
PER_W = N_IN // N_WORKERS
CHUNK = 800
N_CHUNKS = PER_W // CHUNK
VECS = CHUNK // 16
OUT_CH = 32


def _decode_16(X, iota32):
    hi_nz = ((X & 0xFF00) != 0).astype(jnp.int32)
    lo_nz = ((X & 0x00FF) != 0).astype(jnp.int32)
    outs = []
    for c in range(8):
        outs.append(((X >> (15 - c)) & 1).astype(jnp.float32))
    for c in range(8):
        outs.append((((X >> (15 - c)) & 1) ^ hi_nz).astype(jnp.float32))
    for c in range(8):
        outs.append(((X >> (7 - c)) & 1).astype(jnp.float32))
    for c in range(8):
        outs.append((((X >> (7 - c)) & 1) ^ lo_nz).astype(jnp.float32))
    return outs


def _body(x_hbm, out_hbm, in_v, out_v):
    cid = lax.axis_index("c")
    sid = lax.axis_index("s")
    wid = sid * 2 + cid
    base = wid * PER_W
    iota = lax.iota(jnp.int32, 16)
    iota32 = iota * OUT_CH

    def chunk(g, carry):
        in_off = base + g * CHUNK
        pltpu.sync_copy(x_hbm.at[pl.ds(in_off, CHUNK)], in_v)

        def inner(i, c2):
            X = in_v[pl.ds(i * 16, 16)]
            ov = out_v.at[pl.ds(i * (16 * OUT_CH), 16 * OUT_CH)]
            chans = _decode_16(X, iota32)
            for c in range(OUT_CH):
                plsc.store_scatter(ov, [iota32 + c], chans[c])
            return c2

        lax.fori_loop(0, VECS, inner, 0, unroll=False)
        pltpu.sync_copy(out_v, out_hbm.at[pl.ds(in_off * OUT_CH, CHUNK * OUT_CH)])
        return carry

    lax.fori_loop(0, N_CHUNKS, chunk, 0, unroll=False)


@functools.partial(jax.jit, static_argnames=())
def _sc_decode(x_flat):
    mesh = plsc.VectorSubcoreMesh(core_axis_name="c", subcore_axis_name="s")
    fn = functools.partial(
        pl.kernel,
        out_type=jax.ShapeDtypeStruct((N_IN * OUT_CH,), jnp.float32),
        mesh=mesh,
        scratch_types=[
            pltpu.VMEM((CHUNK,), jnp.int32),
            pltpu.VMEM((CHUNK * OUT_CH,), jnp.float32),
        ],
        compiler_params=pltpu.CompilerParams(needs_layout_passes=False),
    )(_body)
    return fn(x_flat)


def kernel(x, mask):
    del mask
    out_flat = _sc_decode(x.reshape(-1))
    return out_flat.reshape(BATCH, FIELDS, OUT_CH)


# speedup vs baseline: 9.6864x; 9.6864x over previous
import functools

import jax
import jax.numpy as jnp
from jax import lax
from jax.experimental import pallas as pl
from jax.experimental.pallas import tpu as pltpu, tpu_sc as plsc

BATCH = 16384
FIELDS = 100
OUT_CH = 32
N_WORKERS = 32
BW = BATCH // N_WORKERS


def _decode_item(row_ref, out_ref):
    for k in range(BW // 16):
        X = row_ref[pl.ds(k * 16, 16)]
        hi_nz = ((X & 0xFF00) != 0).astype(jnp.int32)
        lo_nz = ((X & 0x00FF) != 0).astype(jnp.int32)
        sl = pl.ds(k * 16, 16)
        for c in range(8):
            bh = (X >> (15 - c)) & 1
            bl = (X >> (7 - c)) & 1
            out_ref[c, sl] = bh.astype(jnp.float32)
            out_ref[8 + c, sl] = (bh ^ hi_nz).astype(jnp.float32)
            out_ref[16 + c, sl] = bl.astype(jnp.float32)
            out_ref[24 + c, sl] = (bl ^ lo_nz).astype(jnp.float32)


def _body(xt_hbm, out_hbm, row0, row1, tile0, tile1, isem0, isem1, osem0, osem1):
    wid = lax.axis_index("s") * 2 + lax.axis_index("c")
    b0 = wid * BW
    rows = (row0, row1)
    tiles = (tile0, tile1)
    isems = (isem0, isem1)
    osems = (osem0, osem1)

    def src_at(f):
        return xt_hbm.at[pl.ds(f * BATCH + b0, BW)]

    pltpu.async_copy(src_at(0), row0, isem0)

    def step(t2, carry):
        for p in range(2):
            f = t2 * 2 + p
            pltpu.make_async_copy(src_at(f), rows[p], isems[p]).wait()

            @pl.when(f + 1 < FIELDS)
            def _():
                pltpu.async_copy(src_at(f + 1), rows[1 - p], isems[1 - p])

            @pl.when(t2 >= 1)
            def _():
                pltpu.make_async_copy(
                    tiles[p], out_hbm.at[0, :, pl.ds(0, BW)], osems[p]
                ).wait()

            _decode_item(rows[p], tiles[p])

            pltpu.async_copy(
                tiles[p], out_hbm.at[f, :, pl.ds(b0, BW)], osems[p]
            )
        return carry

    lax.fori_loop(0, FIELDS // 2, step, 0, unroll=False)
    for p in range(2):
        pltpu.make_async_copy(
            tiles[p], out_hbm.at[0, :, pl.ds(0, BW)], osems[p]
        ).wait()


@jax.jit
def _sc_decode(xt_flat):
    mesh = plsc.VectorSubcoreMesh(core_axis_name="c", subcore_axis_name="s")
    fn = functools.partial(
        pl.kernel,
        out_type=jax.ShapeDtypeStruct((FIELDS, OUT_CH, BATCH), jnp.float32),
        mesh=mesh,
        scratch_types=[
            pltpu.VMEM((BW,), jnp.int32),
            pltpu.VMEM((BW,), jnp.int32),
            pltpu.VMEM((OUT_CH, BW), jnp.float32),
            pltpu.VMEM((OUT_CH, BW), jnp.float32),
            pltpu.SemaphoreType.DMA,
            pltpu.SemaphoreType.DMA,
            pltpu.SemaphoreType.DMA,
            pltpu.SemaphoreType.DMA,
        ],
        compiler_params=pltpu.CompilerParams(needs_layout_passes=False),
    )(_body)
    return fn(xt_flat)


def kernel(x, mask):
    del mask
    xt_flat = x.T.reshape(-1)
    out_fcb = _sc_decode(xt_flat)
    return jnp.transpose(out_fcb, (2, 0, 1))
